# Initial kernel scaffold; baseline (speedup 1.0000x reference)
import jax
import jax.numpy as jnp
from jax.experimental import pallas as pl


def kernel(x, values):
    raise NotImplementedError("write your pallas kernel here")


# speedup vs baseline: 3.2912x; 3.2912x over previous
import functools

import jax
import jax.numpy as jnp
from jax import lax
from jax.experimental import pallas as pl
from jax.experimental.pallas import tpu as pltpu
from jax.experimental.pallas import tpu_sc as plsc

N_DIMS = 3
N_FIELDS = 64
RES = 64
GRID = RES + 1
N_POINTS = 131072

NC = 2
NS = 16
L = 16
NW = NC * NS
PTS_PER_W = N_POINTS // NW
C = 32
NCHUNK = PTS_PER_W // C
NIDX = 8 * C
SEG = 128
NSEG = NIDX // SEG


def _sc_kernel(xt_hbm, tab_hbm, out_hbm, xv, idxv, rows, outv, sem):
    wid = lax.axis_index("s") * NC + lax.axis_index("c")
    base = wid * PTS_PER_W

    def chunk_body(t, carry):
        p0 = base + t * C
        for d in range(N_DIMS):
            pltpu.sync_copy(
                xt_hbm.at[pl.ds(d * N_POINTS + p0, C)],
                xv.at[pl.ds(d * C, C)],
            )
        frs = []
        for g in range(C // L):
            fl = []
            fr = []
            for d in range(N_DIMS):
                td = xv[pl.ds(d * C + g * L, L)] * float(RES)
                fld = td.astype(jnp.int32)
                fl.append(fld)
                fr.append(td - fld.astype(jnp.float32))
            frs.append(fr)
            flat = (fl[0] * GRID + fl[1]) * GRID + fl[2]
            for c in range(8):
                dx, dy, dz = (c >> 2) & 1, (c >> 1) & 1, c & 1
                cidx = flat + (dx * GRID * GRID + dy * GRID + dz)
                j = c * C + g * L
                idxv[j // SEG, pl.ds(j % SEG, L)] = cidx
        cps = []
        for s in range(NSEG):
            cps.append(
                pltpu.async_copy(
                    tab_hbm.at[idxv.at[s]],
                    rows.at[pl.ds(s * SEG, SEG)],
                    sem,
                )
            )
        for cp in cps:
            cp.wait()

        for g in range(C // L):
            fr = frs[g]
            for i in range(L):
                p = g * L + i
                fx = fr[0][i]
                fy = fr[1][i]
                fz = fr[2][i]
                for r in range(N_FIELDS // L):
                    sl = pl.ds(r * L, L)
                    v = [rows[c * C + p, sl] for c in range(8)]
                    a00 = v[0] + (v[1] - v[0]) * fz
                    a01 = v[2] + (v[3] - v[2]) * fz
                    a10 = v[4] + (v[5] - v[4]) * fz
                    a11 = v[6] + (v[7] - v[6]) * fz
                    b0 = a00 + (a01 - a00) * fy
                    b1 = a10 + (a11 - a10) * fy
                    outv[p, sl] = b0 + (b1 - b0) * fx
        pltpu.sync_copy(outv, out_hbm.at[pl.ds(p0, C)])
        return carry

    lax.fori_loop(0, NCHUNK, chunk_body, 0)


@jax.jit
def kernel(x, values):
    xt = x.T.reshape(-1)
    tab = values.reshape(GRID * GRID * GRID, N_FIELDS)
    run = functools.partial(
        pl.kernel,
        out_type=jax.ShapeDtypeStruct((N_POINTS, N_FIELDS), jnp.float32),
        mesh=plsc.VectorSubcoreMesh(core_axis_name="c", subcore_axis_name="s"),
        compiler_params=pltpu.CompilerParams(use_tc_tiling_on_sc=False),
        scratch_types=[
            pltpu.VMEM((N_DIMS * C,), jnp.float32),
            pltpu.VMEM((NSEG, SEG), jnp.int32),
            pltpu.VMEM((NIDX, N_FIELDS), jnp.float32),
            pltpu.VMEM((C, N_FIELDS), jnp.float32),
            pltpu.SemaphoreType.DMA,
        ],
    )(_sc_kernel)
    return run(xt, tab)


# speedup vs baseline: 4.4479x; 1.3514x over previous
import functools

import jax
import jax.numpy as jnp
from jax import lax
from jax.experimental import pallas as pl
from jax.experimental.pallas import tpu as pltpu
from jax.experimental.pallas import tpu_sc as plsc

N_DIMS = 3
N_FIELDS = 64
RES = 64
GRID = RES + 1
N_POINTS = 131072

NC = 2
NS = 16
L = 16
NW = NC * NS
PTS_PER_W = N_POINTS // NW
C = 32
NCHUNK = PTS_PER_W // C
NIDX = 8 * C
SEG = 128
NSEG = NIDX // SEG


def _sc_kernel(xt_hbm, tab_hbm, out_hbm, xv, idxv, rows, outv,
               gsem0, gsem1, osem0, osem1):
    wid = lax.axis_index("s") * NC + lax.axis_index("c")
    base = wid * PTS_PER_W
    gsems = (gsem0, gsem1)
    osems = (osem0, osem1)

    for d in range(N_DIMS):
        pltpu.sync_copy(
            xt_hbm.at[pl.ds(d * N_POINTS + base, PTS_PER_W)],
            xv.at[pl.ds(d * PTS_PER_W, PTS_PER_W)],
        )

    def coords(t, g):
        fl, fr = [], []
        for d in range(N_DIMS):
            td = xv[pl.ds(d * PTS_PER_W + t * C + g * L, L)] * float(RES)
            fld = td.astype(jnp.int32)
            fl.append(fld)
            fr.append(td - fld.astype(jnp.float32))
        return fl, fr

    def build_and_fire(t, b):
        for g in range(C // L):
            fl, _ = coords(t, g)
            flat = (fl[0] * GRID + fl[1]) * GRID + fl[2]
            for c in range(8):
                dx, dy, dz = (c >> 2) & 1, (c >> 1) & 1, c & 1
                cidx = flat + (dx * GRID * GRID + dy * GRID + dz)
                j = c * C + g * L
                idxv[b, j // SEG, pl.ds(j % SEG, L)] = cidx
        for s in range(NSEG):
            pltpu.async_copy(
                tab_hbm.at[idxv.at[b, s]],
                rows.at[b, pl.ds(s * SEG, SEG)],
                gsems[b],
            )

    def drain_gather(b):
        for s in range(NSEG):
            pltpu.make_async_copy(
                tab_hbm.at[idxv.at[b, s]],
                rows.at[b, pl.ds(s * SEG, SEG)],
                gsems[b],
            ).wait()

    def compute(t, b):
        for g in range(C // L):
            _, fr = coords(t, g)
            for i in range(L):
                p = g * L + i
                fx = fr[0][i]
                fy = fr[1][i]
                fz = fr[2][i]
                for r in range(N_FIELDS // L):
                    sl = pl.ds(r * L, L)
                    v = [rows[b, c * C + p, sl] for c in range(8)]
                    a00 = v[0] + (v[1] - v[0]) * fz
                    a01 = v[2] + (v[3] - v[2]) * fz
                    a10 = v[4] + (v[5] - v[4]) * fz
                    a11 = v[6] + (v[7] - v[6]) * fz
                    b0 = a00 + (a01 - a00) * fy
                    b1 = a10 + (a11 - a10) * fy
                    outv[b, p, sl] = b0 + (b1 - b0) * fx

    def fire_out(t, b):
        pltpu.async_copy(outv.at[b], out_hbm.at[pl.ds(base + t * C, C)], osems[b])

    def drain_out(t, b):
        pltpu.make_async_copy(
            outv.at[b], out_hbm.at[pl.ds(base + t * C, C)], osems[b]
        ).wait()

    build_and_fire(0, 0)
    build_and_fire(1, 1)

    def pair_body(t2, carry):
        t = 2 * t2
        for b in range(2):
            tb = t + b

            @pl.when(t2 > 0)
            def _():
                drain_out(tb - 2, b)

            drain_gather(b)
            compute(tb, b)
            fire_out(tb, b)

            @pl.when(t2 < NCHUNK // 2 - 1)
            def _():
                build_and_fire(tb + 2, b)

        return carry

    lax.fori_loop(0, NCHUNK // 2, pair_body, 0)
    drain_out(NCHUNK - 2, 0)
    drain_out(NCHUNK - 1, 1)


@jax.jit
def kernel(x, values):
    xt = x.T.reshape(-1)
    tab = values.reshape(GRID * GRID * GRID, N_FIELDS)
    run = functools.partial(
        pl.kernel,
        out_type=jax.ShapeDtypeStruct((N_POINTS, N_FIELDS), jnp.float32),
        mesh=plsc.VectorSubcoreMesh(core_axis_name="c", subcore_axis_name="s"),
        compiler_params=pltpu.CompilerParams(use_tc_tiling_on_sc=False),
        scratch_types=[
            pltpu.VMEM((N_DIMS * PTS_PER_W,), jnp.float32),
            pltpu.VMEM((2, NSEG, SEG), jnp.int32),
            pltpu.VMEM((2, NIDX, N_FIELDS), jnp.float32),
            pltpu.VMEM((2, C, N_FIELDS), jnp.float32),
            pltpu.SemaphoreType.DMA,
            pltpu.SemaphoreType.DMA,
            pltpu.SemaphoreType.DMA,
            pltpu.SemaphoreType.DMA,
        ],
    )(_sc_kernel)
    return run(xt, tab)
